# baseline (speedup 1.0000x reference)
import functools

import jax
import jax.numpy as jnp
from jax import lax
from jax.experimental import pallas as pl
from jax.experimental.pallas import tpu as pltpu
from jax.experimental.pallas import tpu_sc as plsc

B, LS, DM = 4096, 200, 64
DP = 128
TOT = B * LS
V = 1000000
NC, NS = 2, 16
NW = NC * NS
PER_W = TOT // NW
CHUNK = 512
NCHUNK = PER_W // CHUNK

RB = 128
NBLK = V // RB
TAIL = V - NBLK * RB

_mesh = plsc.VectorSubcoreMesh(core_axis_name="c", subcore_axis_name="s")


def _transpose_block(src_v, stg_v, nrow16, row0):
    lanes = lax.iota(jnp.int32, 16)
    rows = [c * 16 + lanes for c in range(nrow16)]
    for f in range(DM):
        col_lo = jnp.full((16,), f, jnp.int32)
        col_hi = jnp.full((16,), f + DM, jnp.int32)
        for c in range(nrow16):
            vals = src_v[f, pl.ds(row0 + c * 16, 16)]
            plsc.store_scatter(stg_v, [rows[c], col_lo], vals)
            plsc.store_scatter(stg_v, [rows[c], col_hi], vals)


@functools.partial(
    pl.kernel,
    out_type=jax.ShapeDtypeStruct((V, DP), jnp.float32),
    mesh=_mesh,
    scratch_types=[
        pltpu.VMEM((DM, RB), jnp.float32),
        pltpu.VMEM((RB, DP), jnp.float32),
        pltpu.VMEM((DM, TAIL), jnp.float32),
    ],
    compiler_params=pltpu.CompilerParams(needs_layout_passes=False),
)
def _sc_relayout(tt_hbm, tp_hbm, src_v, stg_v, tail_v):
    wid = lax.axis_index("s") * NC + lax.axis_index("c")
    nblk_w = (NBLK + NW - 1) // NW

    def body(i, carry):
        t = wid + i * NW

        @pl.when(t < NBLK)
        def _():
            off = pl.multiple_of(t * RB, RB)
            pltpu.sync_copy(tt_hbm.at[:, pl.ds(off, RB)], src_v)
            _transpose_block(src_v, stg_v, RB // 16, 0)
            pltpu.sync_copy(stg_v, tp_hbm.at[pl.ds(off, RB)])

        return carry

    lax.fori_loop(0, nblk_w, body, 0)

    @pl.when(wid == 0)
    def _():
        pltpu.sync_copy(tt_hbm.at[:, pl.ds(NBLK * RB, TAIL)], tail_v)
        _transpose_block(tail_v, stg_v, TAIL // 16, 0)
        pltpu.sync_copy(
            stg_v.at[pl.ds(0, TAIL)], tp_hbm.at[pl.ds(NBLK * RB, TAIL)]
        )


@functools.partial(
    pl.kernel,
    out_type=jax.ShapeDtypeStruct((TOT, DP), jnp.float32),
    mesh=_mesh,
    scratch_types=[
        pltpu.VMEM((PER_W,), jnp.int32),
        pltpu.VMEM((CHUNK, DP), jnp.float32),
        pltpu.SemaphoreType.DMA,
    ],
)
def _sc_gather(idx_hbm, table_hbm, out_hbm, idx_v, rows_v, gsem):
    wid = lax.axis_index("s") * NC + lax.axis_index("c")
    base = wid * PER_W
    pltpu.sync_copy(idx_hbm.at[pl.ds(base, PER_W)], idx_v)

    def body(i, carry):
        off = i * CHUNK
        pltpu.async_copy(
            table_hbm.at[idx_v.at[pl.ds(off, CHUNK)]], rows_v, gsem
        ).wait()
        pltpu.sync_copy(rows_v, out_hbm.at[pl.ds(base + off, CHUNK)])
        return carry

    lax.fori_loop(0, NCHUNK, body, 0)


def kernel(source, table):
    idx = source.T.reshape(TOT).astype(jnp.int32)
    tpack = _sc_relayout(table.T)
    out = _sc_gather(idx, tpack)
    return out.reshape(LS, B, DP)[:, :, :DM].transpose(1, 0, 2)


# speedup vs baseline: 1.9214x; 1.9214x over previous
import functools

import jax
import jax.numpy as jnp
from jax import lax
from jax.experimental import pallas as pl
from jax.experimental.pallas import tpu as pltpu
from jax.experimental.pallas import tpu_sc as plsc

B, LS, DM = 4096, 200, 64
DP = 128
TOT = B * LS
V = 1000000
NC, NS = 2, 16
NW = NC * NS
PER_W = TOT // NW
CHUNK = 512
NCHUNK = PER_W // CHUNK

RB = 128
NBLK = V // RB
TAIL = V - NBLK * RB

_mesh = plsc.VectorSubcoreMesh(core_axis_name="c", subcore_axis_name="s")


def _transpose_block(src_v, stg_v, nrow16, row0):
    lanes = lax.iota(jnp.int32, 16)
    rows = [c * 16 + lanes for c in range(nrow16)]
    for f in range(DM):
        col_lo = jnp.full((16,), f, jnp.int32)
        for c in range(nrow16):
            vals = src_v[f, pl.ds(row0 + c * 16, 16)]
            plsc.store_scatter(stg_v, [rows[c], col_lo], vals)


@functools.partial(
    pl.kernel,
    out_type=jax.ShapeDtypeStruct((V, DP), jnp.float32),
    mesh=_mesh,
    scratch_types=[
        pltpu.VMEM((2, DM, RB), jnp.float32),
        pltpu.VMEM((2, RB, DP), jnp.float32),
        pltpu.VMEM((DM, TAIL), jnp.float32),
        pltpu.SemaphoreType.DMA,
        pltpu.SemaphoreType.DMA,
        pltpu.SemaphoreType.DMA,
        pltpu.SemaphoreType.DMA,
    ],
    compiler_params=pltpu.CompilerParams(needs_layout_passes=False),
)
def _sc_relayout(tt_hbm, tp_hbm, src_v, stg_v, tail_v, is0, is1, os0, os1):
    wid = lax.axis_index("s") * NC + lax.axis_index("c")
    nblk_w = (NBLK + NW - 1) // NW
    isem = [is0, is1]
    osem = [os0, os1]

    def start_in(i, b):
        t = wid + i * NW

        @pl.when(t < NBLK)
        def _():
            off = pl.multiple_of(t * RB, RB)
            pltpu.async_copy(tt_hbm.at[:, pl.ds(off, RB)], src_v.at[b], isem[b])

    start_in(0, 0)
    start_in(1, 1)

    def body(i2, carry):
        for b in range(2):
            i = i2 * 2 + b
            t = wid + i * NW

            @pl.when(t < NBLK)
            def _(b=b, i=i, t=t):
                off = pl.multiple_of(t * RB, RB)
                pltpu.make_async_copy(
                    tt_hbm.at[:, pl.ds(off, RB)], src_v.at[b], isem[b]
                ).wait()

                @pl.when(i >= 2)
                def _():
                    t2 = wid + (i - 2) * NW
                    off2 = pl.multiple_of(t2 * RB, RB)
                    pltpu.make_async_copy(
                        stg_v.at[b], tp_hbm.at[pl.ds(off2, RB)], osem[b]
                    ).wait()

                _transpose_block(src_v.at[b], stg_v.at[b], RB // 16, 0)
                pltpu.async_copy(
                    stg_v.at[b], tp_hbm.at[pl.ds(off, RB)], osem[b]
                )
                start_in(i + 2, b)

        return carry

    lax.fori_loop(0, (nblk_w + 1) // 2, body, 0)

    for j in (nblk_w - 2, nblk_w - 1):
        t = wid + j * NW

        @pl.when(t < NBLK)
        def _():
            b = j % 2
            off = pl.multiple_of(t * RB, RB)
            pltpu.make_async_copy(
                stg_v.at[b], tp_hbm.at[pl.ds(off, RB)], osem[b]
            ).wait()

    @pl.when(wid == 0)
    def _():
        pltpu.sync_copy(tt_hbm.at[:, pl.ds(NBLK * RB, TAIL)], tail_v)
        _transpose_block(tail_v, stg_v.at[0], TAIL // 16, 0)
        pltpu.sync_copy(
            stg_v.at[0, pl.ds(0, TAIL)], tp_hbm.at[pl.ds(NBLK * RB, TAIL)]
        )


@functools.partial(
    pl.kernel,
    out_type=jax.ShapeDtypeStruct((TOT, DP), jnp.float32),
    mesh=_mesh,
    scratch_types=[
        pltpu.VMEM((PER_W,), jnp.int32),
        pltpu.VMEM((CHUNK, DP), jnp.float32),
        pltpu.SemaphoreType.DMA,
    ],
)
def _sc_gather(idx_hbm, table_hbm, out_hbm, idx_v, rows_v, gsem):
    wid = lax.axis_index("s") * NC + lax.axis_index("c")
    base = wid * PER_W
    pltpu.sync_copy(idx_hbm.at[pl.ds(base, PER_W)], idx_v)

    def body(i, carry):
        off = i * CHUNK
        pltpu.async_copy(
            table_hbm.at[idx_v.at[pl.ds(off, CHUNK)]], rows_v, gsem
        ).wait()
        pltpu.sync_copy(rows_v, out_hbm.at[pl.ds(base + off, CHUNK)])
        return carry

    lax.fori_loop(0, NCHUNK, body, 0)


def kernel(source, table):
    idx = source.T.reshape(TOT).astype(jnp.int32)
    tpack = _sc_relayout(table.T)
    out = _sc_gather(idx, tpack)
    return out.reshape(LS, B, DP)[:, :, :DM].transpose(1, 0, 2)


# speedup vs baseline: 3.3154x; 1.7255x over previous
import functools

import jax
import jax.numpy as jnp
from jax import lax
from jax.experimental import pallas as pl
from jax.experimental.pallas import tpu as pltpu
from jax.experimental.pallas import tpu_sc as plsc

B, LS, DM = 4096, 200, 64
DP = 128
TOT = B * LS
NC, NS = 2, 16
NW = NC * NS
PER_W = TOT // NW
CHUNK = 400
NCHUNK = PER_W // CHUNK

_mesh = plsc.VectorSubcoreMesh(core_axis_name="c", subcore_axis_name="s")


@functools.partial(
    pl.kernel,
    out_type=jax.ShapeDtypeStruct((TOT, DP), jnp.float32),
    mesh=_mesh,
    scratch_types=[
        pltpu.VMEM((PER_W,), jnp.int32),
        pltpu.VMEM((2, CHUNK, DP), jnp.float32),
        pltpu.SemaphoreType.DMA,
        pltpu.SemaphoreType.DMA,
        pltpu.SemaphoreType.DMA,
        pltpu.SemaphoreType.DMA,
    ],
)
def _sc_gather(idx_hbm, table_hbm, out_hbm, idx_v, rows_v, gs0, gs1, os0, os1):
    wid = lax.axis_index("s") * NC + lax.axis_index("c")
    base = wid * PER_W
    gsem = [gs0, gs1]
    osem = [os0, os1]
    pltpu.sync_copy(idx_hbm.at[pl.ds(base, PER_W)], idx_v)

    def start_gather(i, b):
        pltpu.async_copy(
            table_hbm.at[idx_v.at[pl.ds(i * CHUNK, CHUNK)]],
            rows_v.at[b],
            gsem[b],
        )

    start_gather(0, 0)

    def body(i2, carry):
        for b in range(2):
            i = i2 * 2 + b
            off = i * CHUNK
            pltpu.make_async_copy(
                table_hbm.at[idx_v.at[pl.ds(off, CHUNK)]],
                rows_v.at[b],
                gsem[b],
            ).wait()

            @pl.when(i >= 1)
            def _(b=b, i=i):
                off2 = (i - 1) * CHUNK
                pltpu.make_async_copy(
                    rows_v.at[1 - b],
                    out_hbm.at[pl.ds(base + off2, CHUNK)],
                    osem[1 - b],
                ).wait()

            @pl.when(i + 1 < NCHUNK)
            def _(b=b, i=i):
                start_gather(i + 1, 1 - b)

            pltpu.async_copy(
                rows_v.at[b], out_hbm.at[pl.ds(base + off, CHUNK)], osem[b]
            )

        return carry

    lax.fori_loop(0, NCHUNK // 2, body, 0)

    b_last = (NCHUNK - 1) % 2
    pltpu.make_async_copy(
        rows_v.at[b_last],
        out_hbm.at[pl.ds(base + (NCHUNK - 1) * CHUNK, CHUNK)],
        osem[b_last],
    ).wait()


def kernel(source, table):
    idx = source.T.reshape(TOT).astype(jnp.int32)
    tpad = jnp.pad(table, ((0, 0), (0, DP - DM)))
    out = _sc_gather(idx, tpad)
    return out.reshape(LS, B, DP)[:, :, :DM].transpose(1, 0, 2)


# speedup vs baseline: 3.3221x; 1.0020x over previous
import functools

import jax
import jax.numpy as jnp
from jax import lax
from jax.experimental import pallas as pl
from jax.experimental.pallas import tpu as pltpu
from jax.experimental.pallas import tpu_sc as plsc

B, LS, DM = 4096, 200, 64
DP = 128
TOT = B * LS
NC, NS = 2, 16
NW = NC * NS
PER_W = TOT // NW
CHUNK = 320
NCHUNK = PER_W // CHUNK

_mesh = plsc.VectorSubcoreMesh(core_axis_name="c", subcore_axis_name="s")


@functools.partial(
    pl.kernel,
    out_type=jax.ShapeDtypeStruct((TOT, DP), jnp.float32),
    mesh=_mesh,
    scratch_types=[
        pltpu.VMEM((PER_W,), jnp.int32),
        pltpu.VMEM((2, CHUNK, DP), jnp.float32),
        pltpu.SemaphoreType.DMA,
        pltpu.SemaphoreType.DMA,
        pltpu.SemaphoreType.DMA,
        pltpu.SemaphoreType.DMA,
    ],
)
def _sc_gather(idx_hbm, table_hbm, out_hbm, idx_v, rows_v, gs0, gs1, os0, os1):
    wid = lax.axis_index("s") * NC + lax.axis_index("c")
    base = wid * PER_W
    gsem = [gs0, gs1]
    osem = [os0, os1]
    pltpu.sync_copy(idx_hbm.at[pl.ds(base, PER_W)], idx_v)

    def start_gather(i, b):
        pltpu.async_copy(
            table_hbm.at[idx_v.at[pl.ds(i * CHUNK, CHUNK)]],
            rows_v.at[b],
            gsem[b],
        )

    start_gather(0, 0)

    def body(i2, carry):
        for b in range(2):
            i = i2 * 2 + b
            off = i * CHUNK
            pltpu.make_async_copy(
                table_hbm.at[idx_v.at[pl.ds(off, CHUNK)]],
                rows_v.at[b],
                gsem[b],
            ).wait()

            @pl.when(i >= 1)
            def _(b=b, i=i):
                off2 = (i - 1) * CHUNK
                pltpu.make_async_copy(
                    rows_v.at[1 - b],
                    out_hbm.at[pl.ds(base + off2, CHUNK)],
                    osem[1 - b],
                ).wait()

            @pl.when(i + 1 < NCHUNK)
            def _(b=b, i=i):
                start_gather(i + 1, 1 - b)

            pltpu.async_copy(
                rows_v.at[b], out_hbm.at[pl.ds(base + off, CHUNK)], osem[b]
            )

        return carry

    lax.fori_loop(0, NCHUNK // 2, body, 0)

    b_last = (NCHUNK - 1) % 2
    pltpu.make_async_copy(
        rows_v.at[b_last],
        out_hbm.at[pl.ds(base + (NCHUNK - 1) * CHUNK, CHUNK)],
        osem[b_last],
    ).wait()


def kernel(source, table):
    idx = source.T.reshape(TOT).astype(jnp.int32)
    tpad = jnp.pad(table, ((0, 0), (0, DP - DM)))
    out = _sc_gather(idx, tpad)
    return out.reshape(LS, B, DP)[:, :, :DM].transpose(1, 0, 2)
